# baseline (speedup 1.0000x reference)
import functools

import jax
import jax.numpy as jnp
from jax import lax
from jax.experimental import pallas as pl
from jax.experimental.pallas import tpu as pltpu
from jax.experimental.pallas import tpu_sc as plsc

BATCH = 4096
NUM_FIELD = 26
EMB = 32
LANES = 16

NUM_CORES = 2
NUM_SUBCORES = 16
NUM_WORKERS = NUM_CORES * NUM_SUBCORES
BPW = BATCH // NUM_WORKERS
NIDX = BPW * NUM_FIELD
PAD = NIDX + LANES

ROWS_PER_CHUNK = 4
CHUNK_IDX = ROWS_PER_CHUNK * NUM_FIELD
NCHUNK = BPW // ROWS_PER_CHUNK
NFWCHUNK = NIDX // 128

_mesh = plsc.VectorSubcoreMesh(core_axis_name="c", subcore_axis_name="s")


@functools.partial(
    pl.kernel,
    mesh=_mesh,
    out_type=jax.ShapeDtypeStruct((BATCH,), jnp.float32),
    scratch_types=[
        pltpu.VMEM((PAD,), jnp.int32),
        pltpu.VMEM((NIDX,), jnp.int32),
        pltpu.VMEM((PAD,), jnp.float32),
        pltpu.VMEM((PAD,), jnp.float32),
        pltpu.VMEM((CHUNK_IDX, 128), jnp.float32),
        pltpu.VMEM((CHUNK_IDX, 128), jnp.float32),
        pltpu.VMEM((BPW, LANES), jnp.float32),
        pltpu.VMEM((BPW,), jnp.float32),
        pltpu.VMEM((LANES,), jnp.float32),
        pltpu.SemaphoreType.DMA,
        pltpu.SemaphoreType.DMA,
        pltpu.SemaphoreType.DMA,
    ],
    compiler_params=pltpu.CompilerParams(
        needs_layout_passes=False, use_tc_tiling_on_sc=True),
)
def _fm_sc(emb_hbm, fw_hbm, idx_hbm, fv_hbm, bias_hbm, out_hbm,
           idx_v, idxq_v, fv_v, fw_v, buf_a, buf_b, vsum_v, out_v, bias_v,
           sem_a, sem_b, sem_fw):
    wid = lax.axis_index("s") * NUM_CORES + lax.axis_index("c")
    base = wid * NIDX

    pltpu.sync_copy(idx_hbm.at[pl.ds(base, NIDX)], idx_v.at[pl.ds(0, NIDX)])
    pltpu.sync_copy(fv_hbm.at[pl.ds(base, NIDX)], fv_v.at[pl.ds(0, NIDX)])
    pltpu.sync_copy(bias_hbm, bias_v)

    def quarter_body(i, carry):
        idxq_v[pl.ds(i * LANES, LANES)] = (
            idx_v[pl.ds(i * LANES, LANES)] >> 2)
        return carry

    lax.fori_loop(0, NIDX // LANES, quarter_body, 0)

    def fire(c):
        sl = pl.ds(c * CHUNK_IDX, CHUNK_IDX)
        pltpu.async_copy(emb_hbm.at[idxq_v.at[sl]], buf_a, sem_a)

    def fire_b(c):
        sl = pl.ds(c * CHUNK_IDX, CHUNK_IDX)
        pltpu.async_copy(emb_hbm.at[idxq_v.at[sl]], buf_b, sem_b)

    fire(0)
    fire_b(1)

    def fire_fw(c, carry):
        sl = pl.ds(c * 128, 128)
        pltpu.async_copy(fw_hbm.at[idx_v.at[sl]], fw_v.at[sl], sem_fw)
        return carry

    lax.fori_loop(0, NFWCHUNK, fire_fw, 0)

    iota = lax.iota(jnp.int32, LANES)
    m10 = iota < (NUM_FIELD - LANES)
    zeros = jnp.zeros((LANES,), jnp.float32)

    def compute_chunk(c, buf):
        b0 = c * ROWS_PER_CHUNK
        j0 = b0 * NUM_FIELD
        for r in range(ROWS_PER_CHUNK):
            jr = j0 + r * NUM_FIELD
            fvr0 = fv_v[pl.ds(jr, LANES)]
            fvr1 = fv_v[pl.ds(jr + LANES, LANES)]
            offr0 = (idx_v[pl.ds(jr, LANES)] & 3) * EMB
            offr1 = (idx_v[pl.ds(jr + LANES, LANES)] & 3) * EMB
            acc0 = acc1 = sq0 = sq1 = zeros
            for f in range(NUM_FIELD):
                slot = r * NUM_FIELD + f
                if f < LANES:
                    fvs = fvr0[f]
                    off = offr0[f]
                else:
                    fvs = fvr1[f - LANES]
                    off = offr1[f - LANES]
                e0 = buf[slot, pl.ds(off, LANES)]
                e1 = buf[slot, pl.ds(off + LANES, LANES)]
                t0 = e0 * fvs
                t1 = e1 * fvs
                acc0 = acc0 + t0
                acc1 = acc1 + t1
                sq0 = sq0 + t0 * t0
                sq1 = sq1 + t1 * t1
            v = (acc0 * acc0 + acc1 * acc1 - sq0 - sq1) * 0.5
            vsum_v[b0 + r, pl.ds(0, LANES)] = v

    def chunk_body(c, carry):
        sl0 = pl.ds(0, CHUNK_IDX)

        @pl.when(c % 2 == 0)
        def _():
            pltpu.make_async_copy(
                emb_hbm.at[idxq_v.at[sl0]], buf_a, sem_a).wait()
            compute_chunk(c, buf_a)

            @pl.when(c + 2 < NCHUNK)
            def _():
                fire(c + 2)

        @pl.when(c % 2 == 1)
        def _():
            pltpu.make_async_copy(
                emb_hbm.at[idxq_v.at[sl0]], buf_b, sem_b).wait()
            compute_chunk(c, buf_b)

            @pl.when(c + 2 < NCHUNK)
            def _():
                fire_b(c + 2)

        return carry

    lax.fori_loop(0, NCHUNK, chunk_body, 0)

    def drain_fw(c, carry):
        sl = pl.ds(c * 128, 128)
        pltpu.make_async_copy(fw_hbm.at[idx_v.at[sl]], fw_v.at[sl],
                              sem_fw).wait()
        return carry

    lax.fori_loop(0, NFWCHUNK, drain_fw, 0)

    def first_order_body(b, carry):
        j0 = b * NUM_FIELD
        i0 = j0 + iota
        i1 = i0 + LANES
        p0 = plsc.load_gather(fv_v, [i0]) * plsc.load_gather(fw_v, [i0])
        p1 = plsc.load_gather(fv_v, [i1]) * plsc.load_gather(fw_v, [i1])
        vsum_v[b, pl.ds(0, LANES)] = (
            vsum_v[b, pl.ds(0, LANES)] + p0 + jnp.where(m10, p1, 0.0))
        return carry

    lax.fori_loop(0, BPW, first_order_body, 0)

    bias_vec = bias_v[...]

    def red_body(g, carry):
        rb = g * LANES + iota
        y = zeros
        for k in range(LANES):
            col = jnp.full((LANES,), k, jnp.int32)
            y = y + plsc.load_gather(vsum_v, [rb, col])
        x = y + bias_vec
        out_v[pl.ds(g * LANES, LANES)] = 1.0 / (1.0 + jnp.exp(-x))
        return carry

    lax.fori_loop(0, BPW // LANES, red_body, 0)

    pltpu.sync_copy(out_v, out_hbm.at[pl.ds(wid * BPW, BPW)])


def kernel(feat_index, feat_value, first_weights, feat_embeddings, bias):
    idx = feat_index.astype(jnp.int32).reshape(-1)
    fv = feat_value.astype(jnp.float32).reshape(-1)
    fw = first_weights.astype(jnp.float32).reshape(-1)
    emb128 = feat_embeddings.reshape(25000, 128)
    bias_arr = jnp.full((LANES,), bias, jnp.float32)
    out = _fm_sc(emb128, fw, idx, fv, bias_arr)
    return out.reshape(BATCH, 1)


# speedup vs baseline: 1.2354x; 1.2354x over previous
import functools

import jax
import jax.numpy as jnp
from jax import lax
from jax.experimental import pallas as pl
from jax.experimental.pallas import tpu as pltpu
from jax.experimental.pallas import tpu_sc as plsc

BATCH = 4096
NUM_FIELD = 26
EMB = 32
LANES = 16

NUM_CORES = 2
NUM_SUBCORES = 16
NUM_WORKERS = NUM_CORES * NUM_SUBCORES
BPW = BATCH // NUM_WORKERS
NIDX = BPW * NUM_FIELD
NCHUNK = NIDX // 128
PAD = NIDX + LANES

_mesh = plsc.VectorSubcoreMesh(core_axis_name="c", subcore_axis_name="s")

_T_BLK_C = 8192
_T_BLK_R = _T_BLK_C // 4


def _transpose_body(in_ref, out_ref):
    x = in_ref[...]
    y = jnp.transpose(x)
    y3 = y.reshape(_T_BLK_R, 4, EMB)
    for q in range(4):
        out_ref[:, q * EMB:(q + 1) * EMB] = y3[:, q, :]


def _emb_to_lin128(femb_t):
    grid = (100000 + _T_BLK_C - 1) // _T_BLK_C
    return pl.pallas_call(
        _transpose_body,
        grid=(grid,),
        in_specs=[pl.BlockSpec((32, _T_BLK_C), lambda j: (0, j))],
        out_specs=pl.BlockSpec((_T_BLK_R, 128), lambda j: (j, 0)),
        out_shape=jax.ShapeDtypeStruct((25000, 128), jnp.float32),
    )(femb_t)


@functools.partial(
    pl.kernel,
    mesh=_mesh,
    out_type=jax.ShapeDtypeStruct((BATCH,), jnp.float32),
    scratch_types=[
        pltpu.VMEM((NIDX,), jnp.int32),
        pltpu.VMEM((PAD,), jnp.float32),
        pltpu.VMEM((PAD,), jnp.float32),
        pltpu.VMEM((NIDX, EMB), jnp.float32),
        pltpu.VMEM((BPW, LANES), jnp.float32),
        pltpu.VMEM((BPW,), jnp.float32),
        pltpu.VMEM((LANES,), jnp.float32),
        pltpu.SemaphoreType.DMA,
    ],
    compiler_params=pltpu.CompilerParams(
        needs_layout_passes=False, use_tc_tiling_on_sc=False),
)
def _fm_sc(emb_hbm, fw_hbm, idx_hbm, fv_hbm, bias_hbm, out_hbm,
           idx_v, fv_v, fw_v, rows_v, vsum_v, out_v, bias_v, sem):
    wid = lax.axis_index("s") * NUM_CORES + lax.axis_index("c")
    base = wid * NIDX

    pltpu.sync_copy(idx_hbm.at[pl.ds(base, NIDX)], idx_v)
    pltpu.sync_copy(fv_hbm.at[pl.ds(base, NIDX)], fv_v.at[pl.ds(0, NIDX)])
    pltpu.sync_copy(bias_hbm, bias_v)

    copies = []
    for c in range(NCHUNK):
        sl = pl.ds(c * 128, 128)
        copies.append(
            pltpu.async_copy(emb_hbm.at[idx_v.at[sl]], rows_v.at[sl], sem))
        copies.append(
            pltpu.async_copy(fw_hbm.at[idx_v.at[sl]], fw_v.at[sl], sem))
    for cp in copies:
        cp.wait()

    iota = lax.iota(jnp.int32, LANES)
    m10 = iota < (NUM_FIELD - LANES)
    zeros = jnp.zeros((LANES,), jnp.float32)

    def row_body(b, carry):
        j0 = b * NUM_FIELD
        acc0 = acc1 = sq0 = sq1 = zeros
        fvr0 = fv_v[pl.ds(j0, LANES)]
        fvr1 = fv_v[pl.ds(j0 + LANES, LANES)]
        for f in range(NUM_FIELD):
            e0 = rows_v[j0 + f, pl.ds(0, LANES)]
            e1 = rows_v[j0 + f, pl.ds(LANES, LANES)]
            fvs = fvr0[f] if f < LANES else fvr1[f - LANES]
            t0 = e0 * fvs
            t1 = e1 * fvs
            acc0 = acc0 + t0
            acc1 = acc1 + t1
            sq0 = sq0 + t0 * t0
            sq1 = sq1 + t1 * t1
        v = (acc0 * acc0 + acc1 * acc1 - sq0 - sq1) * 0.5
        i0 = j0 + iota
        i1 = i0 + LANES
        p0 = plsc.load_gather(fv_v, [i0]) * plsc.load_gather(fw_v, [i0])
        p1 = plsc.load_gather(fv_v, [i1]) * plsc.load_gather(fw_v, [i1])
        v = v + p0 + jnp.where(m10, p1, 0.0)
        vsum_v[b, pl.ds(0, LANES)] = v
        return carry

    lax.fori_loop(0, BPW, row_body, 0)

    bias_vec = bias_v[...]

    def red_body(g, carry):
        rb = g * LANES + iota
        y = zeros
        for k in range(LANES):
            col = jnp.full((LANES,), k, jnp.int32)
            y = y + plsc.load_gather(vsum_v, [rb, col])
        x = y + bias_vec
        out_v[pl.ds(g * LANES, LANES)] = 1.0 / (1.0 + jnp.exp(-x))
        return carry

    lax.fori_loop(0, BPW // LANES, red_body, 0)

    pltpu.sync_copy(out_v, out_hbm.at[pl.ds(wid * BPW, BPW)])


def kernel(feat_index, feat_value, first_weights, feat_embeddings, bias):
    idx = feat_index.astype(jnp.int32).reshape(-1)
    fv = feat_value.astype(jnp.float32).reshape(-1)
    fw = first_weights.astype(jnp.float32).reshape(-1)
    bias_arr = jnp.full((LANES,), bias, jnp.float32)
    emb_lin = _emb_to_lin128(feat_embeddings.T).reshape(-1).reshape(100000, EMB)
    out = _fm_sc(emb_lin, fw, idx, fv, bias_arr)
    return out.reshape(BATCH, 1)
